# baseline (speedup 1.0000x reference)
import jax
import jax.numpy as jnp
from jax.experimental import pallas as pl
from jax.experimental.pallas import tpu as pltpu


def _fused_body(x_ref, a_ref, o_ref):
    a = a_ref[...]
    support = jnp.dot(a, x_ref[...], preferred_element_type=jnp.float32)
    num_neigh = jnp.sum(a, axis=1, keepdims=True)
    o_ref[...] = support / num_neigh


def kernel(x, adjacency):
    n, d = x.shape
    bi = 128
    grid = (n // bi,)
    return pl.pallas_call(
        _fused_body,
        grid=grid,
        in_specs=[
            pl.BlockSpec((n, d), lambda i: (0, 0)),
            pl.BlockSpec((bi, n), lambda i: (i, 0)),
        ],
        out_specs=pl.BlockSpec((bi, d), lambda i: (i, 0)),
        out_shape=jax.ShapeDtypeStruct((n, d), jnp.float32),
        compiler_params=pltpu.CompilerParams(
            dimension_semantics=("parallel",),
        ),
    )(x, adjacency)


# speedup vs baseline: 1.2221x; 1.2221x over previous
import jax
import jax.numpy as jnp
from jax.experimental import pallas as pl
from jax.experimental.pallas import tpu as pltpu


def _fused_body(x_ref, a_ref, o_ref):
    a = a_ref[...]
    support = jnp.dot(a, x_ref[...], preferred_element_type=jnp.float32)
    num_neigh = jnp.sum(a, axis=1, keepdims=True)
    o_ref[...] = support / num_neigh


def kernel(x, adjacency):
    n, d = x.shape
    bi = 256
    grid = (n // bi,)
    return pl.pallas_call(
        _fused_body,
        grid=grid,
        in_specs=[
            pl.BlockSpec((n, d), lambda i: (0, 0)),
            pl.BlockSpec((bi, n), lambda i: (i, 0)),
        ],
        out_specs=pl.BlockSpec((bi, d), lambda i: (i, 0)),
        out_shape=jax.ShapeDtypeStruct((n, d), jnp.float32),
        compiler_params=pltpu.CompilerParams(
            dimension_semantics=("parallel",),
        ),
    )(x, adjacency)
